# baseline (speedup 1.0000x reference)
import functools

import jax
import jax.numpy as jnp
from jax import lax
from jax.experimental import pallas as pl
from jax.experimental.pallas import tpu as pltpu
from jax.experimental.pallas import tpu_sc as plsc

N = 10000
E = 160000
DC = 144
NACC = 10112
SUBROWS = NACC // 16
KB = 80
EPS = E // 16
NB = EPS // KB

_MESH = plsc.VectorSubcoreMesh(
    core_axis_name="c", subcore_axis_name="s", num_cores=2, num_subcores=16
)


def _make_prop(C, dcw=DC):
    c_per_sc = C // 2

    @functools.partial(
        pl.kernel,
        out_type=jax.ShapeDtypeStruct((NACC, C, dcw), jnp.float32),
        mesh=_MESH,
        scratch_types=[
            pltpu.VMEM((2, KB), jnp.int32),
            pltpu.VMEM((2, KB), jnp.int32),
            pltpu.VMEM((KB, dcw), jnp.float32),
            pltpu.VMEM((KB, dcw), jnp.float32),
            pltpu.VMEM_SHARED((NACC, dcw), jnp.float32),
            pltpu.SemaphoreType.DMA,
            pltpu.SemaphoreType.DMA,
            pltpu.SemaphoreType.DMA,
            pltpu.SemaphoreType.DMA,
            pltpu.SemaphoreType.DMA,
            pltpu.SemaphoreType.DMA,
        ],
        compiler_params=pltpu.CompilerParams(use_tc_tiling_on_sc=False),
    )
    def prop(
        zflat, gidx, zeros, out,
        idx0, idx1, rows0, rows1, acc,
        isem0, isem1, gsem0, gsem1, ssem0, ssem1,
    ):
        c = lax.axis_index("c")
        s = lax.axis_index("s")
        dummy = zeros.at[pl.ds(0, KB)]
        for ci in range(c_per_sc):
            chunk = c * c_per_sc + ci
            base = (chunk * 16 + s) * NB
            pltpu.sync_copy(zeros, acc.at[pl.ds(s * SUBROWS, SUBROWS)])
            plsc.subcore_barrier()

            def body(u, _, base=base):
                @pl.when(u > 0)
                def _():
                    pltpu.make_async_copy(dummy, rows0, ssem0).wait()
                    pltpu.make_async_copy(dummy, rows1, ssem1).wait()

                i0 = pltpu.async_copy(gidx.at[base + 2 * u], idx0, isem0)
                i1 = pltpu.async_copy(gidx.at[base + 2 * u + 1], idx1, isem1)
                i0.wait()
                g0 = pltpu.async_copy(zflat.at[idx0.at[0]], rows0, gsem0)
                i1.wait()
                g1 = pltpu.async_copy(zflat.at[idx1.at[0]], rows1, gsem1)
                g0.wait()
                pltpu.async_copy(rows0, acc.at[idx0.at[1]], ssem0, add=True)
                g1.wait()
                pltpu.async_copy(rows1, acc.at[idx1.at[1]], ssem1, add=True)
                return 0

            lax.fori_loop(0, NB // 2, body, 0)
            pltpu.make_async_copy(dummy, rows0, ssem0).wait()
            pltpu.make_async_copy(dummy, rows1, ssem1).wait()
            pltpu.async_copy(gidx.at[base + NB - 1], idx0, isem0).wait()
            pltpu.async_copy(zflat.at[idx0.at[0]], rows0, gsem0).wait()
            pltpu.sync_copy(rows0, acc.at[idx0.at[1]], add=True)
            plsc.subcore_barrier()
            pltpu.sync_copy(
                acc.at[pl.ds(s * SUBROWS, SUBROWS)],
                out.at[pl.ds(s * SUBROWS, SUBROWS), chunk],
            )

    return prop


_PROP = {4: _make_prop(4), 8: _make_prop(8)}
_DEGPROP = _make_prop(2, dcw=16)


def _matmul(x, w):
    m, k = x.shape
    n = w.shape[1]
    bm = 400
    bk = 384 if k % 384 == 0 else k
    bn = 384 if n % 384 == 0 else n
    grid = (m // bm, n // bn, k // bk)
    nk = grid[2]

    def mm(x_ref, w_ref, o_ref, acc_ref):
        kk = pl.program_id(2)

        @pl.when(kk == 0)
        def _():
            acc_ref[...] = jnp.zeros_like(acc_ref)

        acc_ref[...] += jnp.dot(
            x_ref[...], w_ref[...], preferred_element_type=jnp.float32
        )

        @pl.when(kk == nk - 1)
        def _():
            o_ref[...] = acc_ref[...]

    return pl.pallas_call(
        mm,
        grid=grid,
        in_specs=[
            pl.BlockSpec((bm, bk), lambda i, j, kk: (i, kk)),
            pl.BlockSpec((bk, bn), lambda i, j, kk: (kk, j)),
        ],
        out_specs=pl.BlockSpec((bm, bn), lambda i, j, kk: (i, j)),
        out_shape=jax.ShapeDtypeStruct((m, n), jnp.float32),
        scratch_shapes=[pltpu.VMEM((bm, bn), jnp.float32)],
        compiler_params=pltpu.CompilerParams(
            dimension_semantics=("parallel", "parallel", "arbitrary")
        ),
    )(x, w)


def _edge_ids(gat, sca, C):
    g = gat[None, :] * C + jnp.arange(C, dtype=jnp.int32)[:, None]
    d = jnp.broadcast_to(sca, (C, E))
    a = jnp.stack([g, d], axis=1)
    a = a.reshape(C, 2, 16, NB, KB).transpose(0, 2, 3, 1, 4)
    return a.reshape(C * 16 * NB, 2, KB)


def _cheb_layer(h, dis, gidx, zeros, Ws, bias):
    K, D, _ = Ws.shape
    C = D // DC
    prop = _PROP[C]

    def do_prop(t):
        zflat = (dis[:, None] * t).reshape(N * C, DC)
        mc = prop(zflat, gidx, zeros)
        return mc[:N].reshape(N, D)

    out = bias + _matmul(h, Ws[0])
    tx1 = -dis[:, None] * do_prop(h)
    out = out + _matmul(tx1, Ws[1])
    tx_prev, tx_pp = tx1, h
    for kk in range(2, K):
        tx = -2.0 * dis[:, None] * do_prop(tx_prev) - tx_pp
        out = out + _matmul(tx, Ws[kk])
        tx_pp, tx_prev = tx_prev, tx
    return jnp.maximum(out, 0.0)


def kernel(x, edge_index, W1, b1, W2, b2, W3, b3):
    src = edge_index[0]
    dst = edge_index[1]
    mask = src != dst
    trash = jnp.int32(N)
    src2 = jnp.where(mask, src, trash)
    dst2 = jnp.where(mask, dst, trash)
    zeros = jnp.zeros((SUBROWS, DC), jnp.float32)

    degc = _DEGPROP(
        jnp.ones((N * 2, 16), jnp.float32),
        _edge_ids(dst, src2, 2),
        jnp.zeros((SUBROWS, 16), jnp.float32),
    )
    deg = degc[:N, 0, 0]
    dis = jnp.where(deg > 0, lax.rsqrt(jnp.maximum(deg, 1.0)), 0.0)

    gidx8 = _edge_ids(src, dst2, 8)
    gidx4 = _edge_ids(src, dst2, 4)

    h = _cheb_layer(x, dis, gidx8, zeros, W1, b1)
    h = _cheb_layer(h, dis, gidx8, zeros, W2, b2)
    h = _cheb_layer(h, dis, gidx4, zeros, W3, b3)
    return h


# speedup vs baseline: 1.3792x; 1.3792x over previous
import functools

import jax
import jax.numpy as jnp
from jax import lax
from jax.experimental import pallas as pl
from jax.experimental.pallas import tpu as pltpu
from jax.experimental.pallas import tpu_sc as plsc

N = 10000
E = 160000
DC = 128
NACC = 10112
SUBROWS = NACC // 16
KB = 80
EPS = E // 16
NB = EPS // KB

_MESH = plsc.VectorSubcoreMesh(
    core_axis_name="c", subcore_axis_name="s", num_cores=2, num_subcores=16
)


def _make_prop(C):
    cpc = (C + 1) // 2

    @functools.partial(
        pl.kernel,
        out_type=jax.ShapeDtypeStruct((C, NACC, DC), jnp.float32),
        mesh=_MESH,
        scratch_types=[
            pltpu.VMEM((1, EPS), jnp.int32),
            pltpu.VMEM((1, EPS), jnp.int32),
            pltpu.VMEM((KB,), jnp.int32),
            pltpu.VMEM((KB,), jnp.int32),
            pltpu.VMEM((KB,), jnp.int32),
            pltpu.VMEM((KB,), jnp.int32),
            pltpu.VMEM((KB, DC), jnp.float32),
            pltpu.VMEM((KB, DC), jnp.float32),
            pltpu.VMEM_SHARED((NACC, DC), jnp.float32),
            pltpu.SemaphoreType.DMA,
            pltpu.SemaphoreType.DMA,
            pltpu.SemaphoreType.DMA,
            pltpu.SemaphoreType.DMA,
        ],
    )
    def prop(
        zflat, srcr, dstr, zeros, out,
        src_v, dst_v, sidx0, sidx1, didx0, didx1, rows0, rows1, acc,
        gsem0, gsem1, ssem0, ssem1,
    ):
        c = lax.axis_index("c")
        s = lax.axis_index("s")
        dummy = zeros.at[pl.ds(0, KB)]
        pltpu.sync_copy(srcr.at[s], src_v)
        pltpu.sync_copy(dstr.at[s], dst_v)

        def build(b, off, sidx, didx):
            for j in range(KB // 16):
                sl = pl.ds(b * KB + j * 16, 16)
                d = pl.ds(j * 16, 16)
                sidx[d] = src_v[0, sl] + off
                didx[d] = dst_v[0, sl]

        for ci in range(cpc):
            chunk = c * cpc + ci

            def chunk_body(chunk=chunk):
                off = chunk * N
                pltpu.sync_copy(zeros, acc.at[pl.ds(s * SUBROWS, SUBROWS)])
                plsc.subcore_barrier()

                def body(u, _, off=off):
                    @pl.when(u > 0)
                    def _():
                        pltpu.make_async_copy(dummy, rows0, ssem0).wait()
                        pltpu.make_async_copy(dummy, rows1, ssem1).wait()

                    build(2 * u, off, sidx0, didx0)
                    g0 = pltpu.async_copy(zflat.at[sidx0], rows0, gsem0)
                    build(2 * u + 1, off, sidx1, didx1)
                    g1 = pltpu.async_copy(zflat.at[sidx1], rows1, gsem1)
                    g0.wait()
                    pltpu.async_copy(rows0, acc.at[didx0], ssem0, add=True)
                    g1.wait()
                    pltpu.async_copy(rows1, acc.at[didx1], ssem1, add=True)
                    return 0

                lax.fori_loop(0, NB // 2, body, 0)
                pltpu.make_async_copy(dummy, rows0, ssem0).wait()
                pltpu.make_async_copy(dummy, rows1, ssem1).wait()
                build(NB - 1, off, sidx0, didx0)
                pltpu.async_copy(zflat.at[sidx0], rows0, gsem0).wait()
                pltpu.sync_copy(rows0, acc.at[didx0], add=True)
                plsc.subcore_barrier()
                pltpu.sync_copy(
                    acc.at[pl.ds(s * SUBROWS, SUBROWS)],
                    out.at[chunk, pl.ds(s * SUBROWS, SUBROWS)],
                )

            if ci < C - cpc:
                chunk_body()
            else:
                pl.when(c == 0)(chunk_body)

    return prop


_PROP = {2: _make_prop(2), 5: _make_prop(5), 9: _make_prop(9)}


def _matmul(x, w, c2):
    C = x.shape[0]
    bm = 400
    cb = 3 if c2 % 3 == 0 else c2
    bn = cb * DC
    grid = (N // bm, c2 // cb, C)

    def mm(x_ref, w_ref, o_ref, acc_ref):
        kk = pl.program_id(2)

        @pl.when(kk == 0)
        def _():
            acc_ref[...] = jnp.zeros_like(acc_ref)

        acc_ref[...] += jnp.dot(
            x_ref[0], w_ref[0], preferred_element_type=jnp.float32
        )

        @pl.when(kk == C - 1)
        def _():
            for ch in range(cb):
                o_ref[ch] = acc_ref[:, ch * DC:(ch + 1) * DC]

    return pl.pallas_call(
        mm,
        grid=grid,
        in_specs=[
            pl.BlockSpec((1, bm, DC), lambda i, j, kk: (kk, i, 0)),
            pl.BlockSpec((1, DC, bn), lambda i, j, kk: (kk, 0, j)),
        ],
        out_specs=pl.BlockSpec((cb, bm, DC), lambda i, j, kk: (j, i, 0)),
        out_shape=jax.ShapeDtypeStruct((c2, N, DC), jnp.float32),
        scratch_shapes=[pltpu.VMEM((bm, bn), jnp.float32)],
        compiler_params=pltpu.CompilerParams(
            dimension_semantics=("parallel", "parallel", "arbitrary")
        ),
    )(x, w)


def _cheb_layer(h, dis, srcr, dstr, zeros, Ws, bias, c2):
    K = Ws.shape[0]
    C = h.shape[0]
    prop = _PROP[C]
    disb = dis[None, :, None]

    def do_prop(t):
        mc = prop((disb * t).reshape(C * N, DC), srcr, dstr, zeros)
        return mc[:, :N, :]

    def wk(k):
        return Ws[k].reshape(C, DC, c2 * DC)

    out = bias + _matmul(h, wk(0), c2)
    tx1 = -disb * do_prop(h)
    out = out + _matmul(tx1, wk(1), c2)
    tx_prev, tx_pp = tx1, h
    for k in range(2, K):
        tx = -2.0 * disb * do_prop(tx_prev) - tx_pp
        out = out + _matmul(tx, wk(k), c2)
        tx_pp, tx_prev = tx_prev, tx
    return jnp.maximum(out, 0.0)


def _pad_w(Ws, din_pad, dout_pad):
    K, din, dout = Ws.shape
    return jnp.pad(Ws, ((0, 0), (0, din_pad - din), (0, dout_pad - dout)))


def _pad_b(b, dout_pad):
    return jnp.pad(b, (0, dout_pad - b.shape[0])).reshape(-1, 1, DC)


def kernel(x, edge_index, W1, b1, W2, b2, W3, b3):
    src = edge_index[0]
    dst = edge_index[1]
    mask = src != dst
    trash = jnp.int32(N)
    src2 = jnp.where(mask, src, trash)
    dst2 = jnp.where(mask, dst, trash)
    zeros = jnp.zeros((SUBROWS, DC), jnp.float32)

    srcr = src.reshape(16, 1, EPS)
    dstr = dst2.reshape(16, 1, EPS)

    degc = _PROP[2](
        jnp.ones((N * 2, DC), jnp.float32),
        dst.reshape(16, 1, EPS),
        src2.reshape(16, 1, EPS),
        zeros,
    )
    deg = degc[0, :N, 0]
    dis = jnp.where(deg > 0, lax.rsqrt(jnp.maximum(deg, 1.0)), 0.0)

    xcm = x.reshape(N, 9, DC).transpose(1, 0, 2)
    h = _cheb_layer(xcm, dis, srcr, dstr, zeros, W1, _pad_b(b1, 1152), 9)
    h = _cheb_layer(h, dis, srcr, dstr, zeros,
                    _pad_w(W2, 1152, 640), _pad_b(b2, 640), 5)
    h = _cheb_layer(h, dis, srcr, dstr, zeros,
                    _pad_w(W3, 640, 384), _pad_b(b3, 384), 3)
    return h.transpose(1, 0, 2).reshape(N, 384)[:, :288]


# speedup vs baseline: 1.3884x; 1.0066x over previous
import functools

import jax
import jax.numpy as jnp
from jax import lax
from jax.experimental import pallas as pl
from jax.experimental.pallas import tpu as pltpu
from jax.experimental.pallas import tpu_sc as plsc

N = 10000
E = 160000
DC = 128
NACC = 10112
SUBROWS = NACC // 16
KB = 80
EPS = E // 16
NB = EPS // KB

_MESH = plsc.VectorSubcoreMesh(
    core_axis_name="c", subcore_axis_name="s", num_cores=2, num_subcores=16
)


def _make_prop(C):
    cpc = (C + 1) // 2

    @functools.partial(
        pl.kernel,
        out_type=jax.ShapeDtypeStruct((C, NACC, DC), jnp.float32),
        mesh=_MESH,
        scratch_types=[
            pltpu.VMEM((1, EPS), jnp.int32),
            pltpu.VMEM((1, EPS), jnp.int32),
            pltpu.VMEM((KB,), jnp.int32),
            pltpu.VMEM((KB,), jnp.int32),
            pltpu.VMEM((KB,), jnp.int32),
            pltpu.VMEM((KB,), jnp.int32),
            pltpu.VMEM((KB, DC), jnp.float32),
            pltpu.VMEM((KB, DC), jnp.float32),
            pltpu.VMEM_SHARED((NACC, DC), jnp.float32),
            pltpu.SemaphoreType.DMA,
            pltpu.SemaphoreType.DMA,
            pltpu.SemaphoreType.DMA,
            pltpu.SemaphoreType.DMA,
        ],
    )
    def prop(
        zflat, srcr, dstr, zeros, out,
        src_v, dst_v, sidx0, sidx1, didx0, didx1, rows0, rows1, acc,
        gsem0, gsem1, ssem0, ssem1,
    ):
        c = lax.axis_index("c")
        s = lax.axis_index("s")
        dummy = zeros.at[pl.ds(0, KB)]
        pltpu.sync_copy(srcr.at[s], src_v)
        pltpu.sync_copy(dstr.at[s], dst_v)

        def build(b, off, sidx, didx):
            for j in range(KB // 16):
                sl = pl.ds(b * KB + j * 16, 16)
                d = pl.ds(j * 16, 16)
                sidx[d] = src_v[0, sl] + off
                didx[d] = dst_v[0, sl]

        for ci in range(cpc):
            chunk = c * cpc + ci

            def chunk_body(chunk=chunk):
                off = chunk * N
                pltpu.sync_copy(zeros, acc.at[pl.ds(s * SUBROWS, SUBROWS)])
                plsc.subcore_barrier()

                def body(u, _, off=off):
                    @pl.when(u > 0)
                    def _():
                        pltpu.make_async_copy(dummy, rows0, ssem0).wait()
                        pltpu.make_async_copy(dummy, rows1, ssem1).wait()

                    build(2 * u, off, sidx0, didx0)
                    g0 = pltpu.async_copy(zflat.at[sidx0], rows0, gsem0)
                    build(2 * u + 1, off, sidx1, didx1)
                    g1 = pltpu.async_copy(zflat.at[sidx1], rows1, gsem1)
                    g0.wait()
                    pltpu.async_copy(rows0, acc.at[didx0], ssem0, add=True)
                    g1.wait()
                    pltpu.async_copy(rows1, acc.at[didx1], ssem1, add=True)
                    return 0

                lax.fori_loop(0, NB // 2, body, 0)
                pltpu.make_async_copy(dummy, rows0, ssem0).wait()
                pltpu.make_async_copy(dummy, rows1, ssem1).wait()
                build(NB - 1, off, sidx0, didx0)
                pltpu.async_copy(zflat.at[sidx0], rows0, gsem0).wait()
                pltpu.sync_copy(rows0, acc.at[didx0], add=True)
                plsc.subcore_barrier()
                pltpu.sync_copy(
                    acc.at[pl.ds(s * SUBROWS, SUBROWS)],
                    out.at[chunk, pl.ds(s * SUBROWS, SUBROWS)],
                )

            if ci < C - cpc:
                chunk_body()
            else:
                pl.when(c == 0)(chunk_body)

    return prop


_PROP = {2: _make_prop(2), 5: _make_prop(5), 9: _make_prop(9)}


def _matmul(x, w, c2):
    C = x.shape[0]
    bm = 400
    cb = 3 if c2 % 3 == 0 else c2
    bn = cb * DC
    grid = (N // bm, c2 // cb, C)

    def mm(x_ref, w_ref, o_ref, acc_ref):
        kk = pl.program_id(2)

        @pl.when(kk == 0)
        def _():
            acc_ref[...] = jnp.zeros_like(acc_ref)

        acc_ref[...] += jnp.dot(
            x_ref[0], w_ref[0], preferred_element_type=jnp.float32
        )

        @pl.when(kk == C - 1)
        def _():
            for ch in range(cb):
                o_ref[ch] = acc_ref[:, ch * DC:(ch + 1) * DC]

    return pl.pallas_call(
        mm,
        grid=grid,
        in_specs=[
            pl.BlockSpec((1, bm, DC), lambda i, j, kk: (kk, i, 0)),
            pl.BlockSpec((1, DC, bn), lambda i, j, kk: (kk, 0, j)),
        ],
        out_specs=pl.BlockSpec((cb, bm, DC), lambda i, j, kk: (j, i, 0)),
        out_shape=jax.ShapeDtypeStruct((c2, N, DC), jnp.float32),
        scratch_shapes=[pltpu.VMEM((bm, bn), jnp.float32)],
        compiler_params=pltpu.CompilerParams(
            dimension_semantics=("parallel", "parallel", "arbitrary")
        ),
    )(x.astype(jnp.bfloat16), w.astype(jnp.bfloat16))


def _cheb_layer(h, dis, srcr, dstr, zeros, Ws, bias, c2):
    K = Ws.shape[0]
    C = h.shape[0]
    prop = _PROP[C]
    disb = dis[None, :, None]

    def do_prop(t):
        mc = prop((disb * t).reshape(C * N, DC), srcr, dstr, zeros)
        return mc[:, :N, :]

    def wk(k):
        return Ws[k].reshape(C, DC, c2 * DC)

    out = bias + _matmul(h, wk(0), c2)
    tx1 = -disb * do_prop(h)
    out = out + _matmul(tx1, wk(1), c2)
    tx_prev, tx_pp = tx1, h
    for k in range(2, K):
        tx = -2.0 * disb * do_prop(tx_prev) - tx_pp
        out = out + _matmul(tx, wk(k), c2)
        tx_pp, tx_prev = tx_prev, tx
    return jnp.maximum(out, 0.0)


def _pad_w(Ws, din_pad, dout_pad):
    K, din, dout = Ws.shape
    return jnp.pad(Ws, ((0, 0), (0, din_pad - din), (0, dout_pad - dout)))


def _pad_b(b, dout_pad):
    return jnp.pad(b, (0, dout_pad - b.shape[0])).reshape(-1, 1, DC)


def kernel(x, edge_index, W1, b1, W2, b2, W3, b3):
    src = edge_index[0]
    dst = edge_index[1]
    mask = src != dst
    trash = jnp.int32(N)
    src2 = jnp.where(mask, src, trash)
    dst2 = jnp.where(mask, dst, trash)
    zeros = jnp.zeros((SUBROWS, DC), jnp.float32)

    srcr = src.reshape(16, 1, EPS)
    dstr = dst2.reshape(16, 1, EPS)

    degc = _PROP[2](
        jnp.ones((N * 2, DC), jnp.float32),
        dst.reshape(16, 1, EPS),
        src2.reshape(16, 1, EPS),
        zeros,
    )
    deg = degc[0, :N, 0]
    dis = jnp.where(deg > 0, lax.rsqrt(jnp.maximum(deg, 1.0)), 0.0)

    xcm = x.reshape(N, 9, DC).transpose(1, 0, 2)
    h = _cheb_layer(xcm, dis, srcr, dstr, zeros, W1, _pad_b(b1, 1152), 9)
    h = _cheb_layer(h, dis, srcr, dstr, zeros,
                    _pad_w(W2, 1152, 640), _pad_b(b2, 640), 5)
    h = _cheb_layer(h, dis, srcr, dstr, zeros,
                    _pad_w(W3, 640, 384), _pad_b(b3, 384), 3)
    return h.transpose(1, 0, 2).reshape(N, 384)[:, :288]


# speedup vs baseline: 1.6554x; 1.1923x over previous
import functools

import jax
import jax.numpy as jnp
from jax import lax
from jax.experimental import pallas as pl
from jax.experimental.pallas import tpu as pltpu
from jax.experimental.pallas import tpu_sc as plsc

N = 10000
E = 160000
DC = 128
NACC = 10112
SUBROWS = NACC // 16
KB = 80
EPS = E // 16
NB = EPS // KB

_MESH = plsc.VectorSubcoreMesh(
    core_axis_name="c", subcore_axis_name="s", num_cores=2, num_subcores=16
)


def _make_prop(C):
    cpc = (C + 1) // 2

    @functools.partial(
        pl.kernel,
        out_type=jax.ShapeDtypeStruct((C, NACC, DC), jnp.float32),
        mesh=_MESH,
        scratch_types=[
            pltpu.VMEM((1, EPS), jnp.int32),
            pltpu.VMEM((1, EPS), jnp.int32),
            pltpu.VMEM((KB,), jnp.int32),
            pltpu.VMEM((KB,), jnp.int32),
            pltpu.VMEM((KB,), jnp.int32),
            pltpu.VMEM((KB,), jnp.int32),
            pltpu.VMEM((KB, DC), jnp.float32),
            pltpu.VMEM((KB, DC), jnp.float32),
            pltpu.VMEM_SHARED((NACC, DC), jnp.float32),
            pltpu.SemaphoreType.DMA,
            pltpu.SemaphoreType.DMA,
            pltpu.SemaphoreType.DMA,
            pltpu.SemaphoreType.DMA,
        ],
    )
    def prop(
        zflat, srcr, dstr, zeros, out,
        src_v, dst_v, sidx0, sidx1, didx0, didx1, rows0, rows1, acc,
        gsem0, gsem1, ssem0, ssem1,
    ):
        c = lax.axis_index("c")
        s = lax.axis_index("s")
        dummy = zeros.at[pl.ds(0, KB)]
        pltpu.sync_copy(srcr.at[s], src_v)
        pltpu.sync_copy(dstr.at[s], dst_v)

        def build(b, off, sidx, didx):
            for j in range(KB // 16):
                sl = pl.ds(b * KB + j * 16, 16)
                d = pl.ds(j * 16, 16)
                sidx[d] = src_v[0, sl] + off
                didx[d] = dst_v[0, sl]

        for ci in range(cpc):
            chunk = c * cpc + ci

            def chunk_body(chunk=chunk):
                off = chunk * N
                pltpu.sync_copy(zeros, acc.at[pl.ds(s * SUBROWS, SUBROWS)])
                plsc.subcore_barrier()

                def body(u, _, off=off):
                    @pl.when(u > 0)
                    def _():
                        pltpu.make_async_copy(dummy, rows0, ssem0).wait()
                        pltpu.make_async_copy(dummy, rows1, ssem1).wait()

                    build(2 * u, off, sidx0, didx0)
                    g0 = pltpu.async_copy(zflat.at[sidx0], rows0, gsem0)
                    build(2 * u + 1, off, sidx1, didx1)
                    g1 = pltpu.async_copy(zflat.at[sidx1], rows1, gsem1)
                    g0.wait()
                    pltpu.async_copy(rows0, acc.at[didx0], ssem0, add=True)
                    g1.wait()
                    pltpu.async_copy(rows1, acc.at[didx1], ssem1, add=True)
                    return 0

                lax.fori_loop(0, NB // 2, body, 0)
                pltpu.make_async_copy(dummy, rows0, ssem0).wait()
                pltpu.make_async_copy(dummy, rows1, ssem1).wait()
                build(NB - 1, off, sidx0, didx0)
                pltpu.async_copy(zflat.at[sidx0], rows0, gsem0).wait()
                pltpu.sync_copy(rows0, acc.at[didx0], add=True)
                plsc.subcore_barrier()
                pltpu.sync_copy(
                    acc.at[pl.ds(s * SUBROWS, SUBROWS)],
                    out.at[chunk, pl.ds(s * SUBROWS, SUBROWS)],
                )

            if ci < C - cpc:
                chunk_body()
            else:
                pl.when(c == 0)(chunk_body)

    return prop


_PROP = {2: _make_prop(2), 5: _make_prop(5), 9: _make_prop(9)}


def _matmul(x, w, c2):
    C = x.shape[0]
    bm = 1000
    cb = 3 if c2 % 3 == 0 else c2
    bn = cb * DC
    kb = 3 if C % 3 == 0 else C
    nk = C // kb
    grid = (N // bm, c2 // cb, nk)

    def mm(x_ref, w_ref, o_ref, acc_ref):
        kk = pl.program_id(2)

        @pl.when(kk == 0)
        def _():
            acc_ref[...] = jnp.zeros_like(acc_ref)

        for ch in range(kb):
            acc_ref[...] += jnp.dot(
                x_ref[ch], w_ref[ch], preferred_element_type=jnp.float32
            )

        @pl.when(kk == nk - 1)
        def _():
            for ch in range(cb):
                o_ref[ch] = acc_ref[:, ch * DC:(ch + 1) * DC]

    return pl.pallas_call(
        mm,
        grid=grid,
        in_specs=[
            pl.BlockSpec((kb, bm, DC), lambda i, j, kk: (kk, i, 0)),
            pl.BlockSpec((kb, DC, bn), lambda i, j, kk: (kk, 0, j)),
        ],
        out_specs=pl.BlockSpec((cb, bm, DC), lambda i, j, kk: (j, i, 0)),
        out_shape=jax.ShapeDtypeStruct((c2, N, DC), jnp.float32),
        scratch_shapes=[pltpu.VMEM((bm, bn), jnp.float32)],
        compiler_params=pltpu.CompilerParams(
            dimension_semantics=("parallel", "parallel", "arbitrary")
        ),
    )(x.astype(jnp.bfloat16), w.astype(jnp.bfloat16))


def _cheb_layer(h, dis, srcr, dstr, zeros, Ws, bias, c2):
    K = Ws.shape[0]
    C = h.shape[0]
    prop = _PROP[C]
    disb = dis[None, :, None]

    def do_prop(t):
        mc = prop((disb * t).reshape(C * N, DC), srcr, dstr, zeros)
        return mc[:, :N, :]

    def wk(k):
        return Ws[k].reshape(C, DC, c2 * DC)

    out = bias + _matmul(h, wk(0), c2)
    tx1 = -disb * do_prop(h)
    out = out + _matmul(tx1, wk(1), c2)
    tx_prev, tx_pp = tx1, h
    for k in range(2, K):
        tx = -2.0 * disb * do_prop(tx_prev) - tx_pp
        out = out + _matmul(tx, wk(k), c2)
        tx_pp, tx_prev = tx_prev, tx
    return jnp.maximum(out, 0.0)


def _pad_w(Ws, din_pad, dout_pad):
    K, din, dout = Ws.shape
    return jnp.pad(Ws, ((0, 0), (0, din_pad - din), (0, dout_pad - dout)))


def _pad_b(b, dout_pad):
    return jnp.pad(b, (0, dout_pad - b.shape[0])).reshape(-1, 1, DC)


def kernel(x, edge_index, W1, b1, W2, b2, W3, b3):
    src = edge_index[0]
    dst = edge_index[1]
    mask = src != dst
    trash = jnp.int32(N)
    src2 = jnp.where(mask, src, trash)
    dst2 = jnp.where(mask, dst, trash)
    zeros = jnp.zeros((SUBROWS, DC), jnp.float32)

    srcr = src.reshape(16, 1, EPS)
    dstr = dst2.reshape(16, 1, EPS)

    degc = _PROP[2](
        jnp.ones((N * 2, DC), jnp.float32),
        dst.reshape(16, 1, EPS),
        src2.reshape(16, 1, EPS),
        zeros,
    )
    deg = degc[0, :N, 0]
    dis = jnp.where(deg > 0, lax.rsqrt(jnp.maximum(deg, 1.0)), 0.0)

    xcm = x.reshape(N, 9, DC).transpose(1, 0, 2)
    h = _cheb_layer(xcm, dis, srcr, dstr, zeros, W1, _pad_b(b1, 1152), 9)
    h = _cheb_layer(h, dis, srcr, dstr, zeros,
                    _pad_w(W2, 1152, 640), _pad_b(b2, 640), 5)
    h = _cheb_layer(h, dis, srcr, dstr, zeros,
                    _pad_w(W3, 640, 384), _pad_b(b3, 384), 3)
    return h.transpose(1, 0, 2).reshape(N, 384)[:, :288]
